# baseline (speedup 1.0000x reference)
import functools

import jax
import jax.numpy as jnp
import numpy as np
from jax import lax
from jax.experimental import pallas as pl
from jax.experimental.pallas import tpu as pltpu
from jax.experimental.pallas import tpu_sc as plsc

VOCAB = 1000000
EMBED_DIM = 64
NOISE_ALPHA = 5.0


_NC, _NS = 2, 16
_NW = _NC * _NS
_CHUNK = 128


def _sc_gather_body(nrows, table_hbm, idx_hbm, out_hbm, idx_v, rows_v, sem):
    wid = lax.axis_index("s") * _NC + lax.axis_index("c")
    rows_per_w = nrows // _NW
    base = wid * rows_per_w

    def chunk(j, carry):
        off = base + j * _CHUNK
        pltpu.sync_copy(idx_hbm.at[pl.ds(off, _CHUNK)], idx_v)
        pltpu.async_copy(table_hbm.at[idx_v], rows_v, sem).wait()
        pltpu.sync_copy(rows_v, out_hbm.at[pl.ds(off, _CHUNK)])
        return carry

    lax.fori_loop(0, rows_per_w // _CHUNK, chunk, 0, unroll=False)


def _sc_gather(table, idx):
    nrows = idx.shape[0]
    mesh = plsc.VectorSubcoreMesh(core_axis_name="c", subcore_axis_name="s")
    return pl.kernel(
        functools.partial(_sc_gather_body, nrows),
        out_type=jax.ShapeDtypeStruct((nrows, EMBED_DIM), jnp.float32),
        mesh=mesh,
        compiler_params=pltpu.CompilerParams(use_tc_tiling_on_sc=False),
        scratch_types=[
            pltpu.VMEM((_CHUNK,), jnp.int32),
            pltpu.VMEM((_CHUNK, EMBED_DIM), jnp.float32),
            pltpu.SemaphoreType.DMA,
        ],
    )(table, idx)


_ROT_A = (13, 15, 26, 6)
_ROT_B = (17, 29, 16, 24)


def _noise_body(mag, cols, block_rows, g_ref, out_ref):
    u32 = jnp.uint32
    base = (pl.program_id(0) * block_rows * cols).astype(u32)
    shape = g_ref.shape
    row = lax.broadcasted_iota(u32, shape, 0)
    col = lax.broadcasted_iota(u32, shape, 1)
    lo = base + row * u32(cols) + col

    ks0 = u32(0)
    ks1 = u32(42)
    ks2 = ks0 ^ ks1 ^ u32(0x1BD11BDA)
    ks = (ks0, ks1, ks2)
    x0 = jnp.full(shape, ks0, dtype=u32)
    x1 = lo + ks1

    def rotl(v, d):
        return (v << u32(d)) | (v >> u32(32 - d))

    for i in range(5):
        rots = _ROT_A if i % 2 == 0 else _ROT_B
        for r in rots:
            x0 = x0 + x1
            x1 = rotl(x1, r)
            x1 = x1 ^ x0
        x0 = x0 + ks[(i + 1) % 3]
        x1 = x1 + ks[(i + 2) % 3] + u32(i + 1)

    bits = x0 ^ x1
    fl = lax.bitcast_convert_type((bits >> u32(9)) | u32(0x3F800000),
                                  jnp.float32) - jnp.float32(1.0)
    noise = jnp.maximum(jnp.float32(-mag),
                        fl * jnp.float32(2.0 * mag) + jnp.float32(-mag))
    out_ref[...] = g_ref[...] + noise


def _tc_noise_add(g, mag):
    n, cols = g.shape
    block_rows = 2048
    grid = n // block_rows
    return pl.pallas_call(
        functools.partial(_noise_body, mag, cols, block_rows),
        out_shape=jax.ShapeDtypeStruct((n, cols), jnp.float32),
        grid=(grid,),
        in_specs=[pl.BlockSpec((block_rows, cols), lambda i: (i, 0))],
        out_specs=pl.BlockSpec((block_rows, cols), lambda i: (i, 0)),
    )(g)


def kernel(x, table):
    b, l = x.shape
    nrows = b * l
    idx = x.reshape(nrows)
    gathered = _sc_gather(table, idx)

    dims = np.float32(l * EMBED_DIM)
    mag = np.float32(NOISE_ALPHA) / np.sqrt(dims)

    flat = gathered.reshape(nrows * EMBED_DIM // 128, 128)
    out = _tc_noise_add(flat, mag)
    return out.reshape(b, l, EMBED_DIM)
